# baseline (speedup 1.0000x reference)
import functools

import numpy as np

import jax
import jax.numpy as jnp
from jax import lax
from jax.experimental import pallas as pl
from jax.experimental.pallas import tpu as pltpu
from jax.experimental.pallas import tpu_sc as plsc

N = 32768
D = 2048
K = 1638
L = 16
NC, NS = 2, 16
NW = NC * NS
ROWS_PER_W = N // NW
CHUNK = 8
CBUF = D + 4 * L
INT_MIN = -2147483648


def _splat(val, dtype):
    return jnp.full((L,), val, dtype)


def _unmap(keys):
    bits = jnp.where(keys >= 0, keys, keys ^ 0x7FFFFFFF)
    return lax.bitcast_convert_type(bits, jnp.float32)


def _body(x_hbm, o_hbm, xbuf, obuf, cbuf1, cbuf2, hist, combo):
    cid = lax.axis_index("c")
    sid = lax.axis_index("s")
    wid = sid * NC + cid
    base_row = wid * ROWS_PER_W
    kk = _splat(K, jnp.int32)
    one = _splat(1, jnp.int32)
    zi = jnp.zeros((L,), jnp.int32)
    zf = jnp.zeros((L,), jnp.float32)
    nan_v = _splat(jnp.nan, jnp.float32)
    ii = lax.iota(jnp.int32, L)

    def zl(l, c3):
        def zc(c, c4):
            hist[l, pl.ds(c * L, L)] = zi
            return c4

        return lax.fori_loop(0, 16, zc, c3)

    lax.fori_loop(0, 16, zl, 0)

    def count_row(r, t):
        @plsc.parallel_loop(0, D, 4 * L, unroll=2, carry=(zi, zi, zi, zi))
        def accs(off, a):
            vs = [xbuf[r, pl.ds(off + j * L, L)] for j in range(4)]
            return tuple(ai + jnp.where(v >= t, one, zi)
                         for ai, v in zip(a, vs))

        return _splat(jnp.sum(sum(accs)), jnp.int32)

    def count_buf(ref, n_pad, t):
        @plsc.parallel_loop(0, n_pad, 2 * L, unroll=2, carry=(zi, zi))
        def accs(off, a):
            vs = [ref[pl.ds(off + j * L, L)] for j in range(2)]
            return tuple(ai + jnp.where(v >= t, one, zi)
                         for ai, v in zip(a, vs))

        return _splat(jnp.sum(sum(accs)), jnp.int32)

    def probe_bits(count_fn, pfx, ff, rr, b_hi, b_lo):
        def rnd(j, state):
            pfx, ff = state
            cand = pfx + (one << (b_hi - j))
            cnt = count_fn(_unmap(cand))
            ok = cnt >= rr
            return jnp.where(ok, cand, pfx), jnp.where(ok, ff, cnt)

        return lax.fori_loop(0, b_hi - b_lo + 1, rnd, (pfx, ff))

    def _compact_group(src, dst, base, off, t_lo, t_hi, width):
        vs = [src(base + j * L) for j in range(width)]
        ms = [(v >= t_lo) & jnp.logical_not(v >= t_hi) for v in vs]
        pcs = [plsc.all_reduce_population_count(m) for m in ms]
        starts = [pcs[0]]
        for j in range(1, width - 1):
            starts.append(starts[-1] + pcs[j])
        plsc.store_compressed(dst.at[pl.ds(off, L)], vs[0], mask=ms[0])
        for j in range(1, width):
            plsc.store_compressed(
                dst.at[pl.ds(off + starts[j - 1][0], L)], vs[j], mask=ms[j])
        return off + (starts[-1] + pcs[-1])[0]

    def compact_from_row(r, t_lo, t_hi):
        def grp(g, off):
            return _compact_group(lambda i: xbuf[r, pl.ds(i, L)], cbuf1,
                                  g * 4 * L, off, t_lo, t_hi, 4)

        n = lax.fori_loop(0, D // (4 * L), grp, np.int32(0))
        for j in range(4):
            cbuf1[pl.ds(n + j * L, L)] = nan_v
        return n

    def compact_from_buf(n_pad, t_lo, t_hi):
        def grp(g, off):
            return _compact_group(lambda i: cbuf1[pl.ds(i, L)], cbuf2,
                                  g * 2 * L, off, t_lo, t_hi, 2)

        n = lax.fori_loop(0, n_pad // (2 * L), grp, np.int32(0))
        cbuf2[pl.ds(n, L)] = nan_v
        cbuf2[pl.ds(n + L, L)] = nan_v
        return n

    def do_chunk(ci, carry):
        row0 = base_row + ci * CHUNK
        pltpu.sync_copy(x_hbm.at[pl.ds(row0, CHUNK), :], xbuf)

        def do_row(r, c2):
            @plsc.parallel_loop(0, D, 4 * L, unroll=2,
                               carry=(zf, zf, zf, zf))
            def sq_accs(off, sq):
                out = []
                for a, j in zip(sq, range(4)):
                    v = xbuf[r, pl.ds(off + j * L, L)]
                    iv = lax.bitcast_convert_type(v, jnp.int32)
                    key = jnp.where(iv >= 0, iv, iv ^ 0x7FFFFFFF)
                    bucket = (key >> 24) + 128
                    plsc.addupdate_scatter(hist, [ii, bucket], one)
                    out.append(a + v * v)
                return tuple(out)

            sv = _splat(jnp.sum(sum(sq_accs)), jnp.float32)
            ib = lax.bitcast_convert_type(sv, jnp.int32)
            y = lax.bitcast_convert_type(0x5F3759DF - (ib >> 1), jnp.float32)
            for _ in range(4):
                y = y * (1.5 - 0.5 * sv * y * y)
            boost = jnp.exp(K * y)

            def comb(c, sums):
                acc = zi
                for l in range(16):
                    acc = acc + hist[l, pl.ds(c * L, L)]
                    hist[l, pl.ds(c * L, L)] = zi
                combo[pl.ds(c * L, L)] = acc
                return sums + jnp.where(ii == c, _splat(jnp.sum(acc),
                                                        jnp.int32), zi)

            sums = lax.fori_loop(0, 16, comb, zi)
            rev = lax.rev(sums, (0,))
            cs = plsc.cumsum(rev)
            j0 = plsc.all_reduce_ffs(cs >= kk)
            b_above = jnp.take_along_axis(cs - rev, j0, axis=0)
            chunk = combo[pl.ds((15 - j0[0]) * L, L)]
            rev2 = lax.rev(chunk, (0,))
            cs2 = plsc.cumsum(rev2)
            j1 = plsc.all_reduce_ffs(b_above + cs2 >= kk)
            bkt = (15 - j0) * L + (15 - j1)
            b_above = b_above + jnp.take_along_axis(cs2 - rev2, j1, axis=0)
            pfx = (bkt - 128) << 24
            rr = kk - b_above

            n1 = compact_from_row(r, _unmap(pfx), _unmap(pfx + (1 << 24)))
            n1_pad = ((n1 + 2 * L - 1) // (2 * L)) * (2 * L)

            pfx, ff = probe_bits(
                lambda t: count_buf(cbuf1, n1_pad, t), pfx, zi, rr, 23, 16)
            rr = rr - ff

            n2 = compact_from_buf(n1_pad, _unmap(pfx),
                                  _unmap(pfx + (1 << 16)))
            n2_pad = ((n2 + 2 * L - 1) // (2 * L)) * (2 * L)

            def l2_sort(_):
                v = cbuf2[pl.ds(0, L)]
                m = lax.iota(jnp.int32, L) < n2
                sk, _sv, _m = plsc.sort_key_val(v, v, mask=m,
                                                descending=True)
                return jnp.take_along_axis(sk, rr - 1, axis=0)

            def l2_probe(_):
                pfx2, _f = probe_bits(
                    lambda t: count_buf(cbuf2, n2_pad, t), pfx, zi, rr,
                    15, 0)
                return _unmap(pfx2)

            t = lax.cond(n2 <= L, l2_sort, l2_probe, 0)

            @plsc.parallel_loop(0, D, 4 * L, unroll=2)
            def mask_store(off):
                for j in range(4):
                    v = xbuf[r, pl.ds(off + j * L, L)]
                    obuf[r, pl.ds(off + j * L, L)] = jnp.where(
                        v >= t, v * boost, 0.0)

            return c2

        carry = lax.fori_loop(0, CHUNK, do_row, carry)
        pltpu.sync_copy(obuf, o_hbm.at[pl.ds(row0, CHUNK), :])
        return carry

    lax.fori_loop(0, ROWS_PER_W // CHUNK, do_chunk, 0)


@jax.jit
def kernel(inputs):
    f = pl.kernel(
        _body,
        out_type=jax.ShapeDtypeStruct((N, D), jnp.float32),
        mesh=plsc.VectorSubcoreMesh(core_axis_name="c", subcore_axis_name="s"),
        compiler_params=pltpu.CompilerParams(needs_layout_passes=False),
        scratch_types=[
            pltpu.VMEM((CHUNK, D), jnp.float32),
            pltpu.VMEM((CHUNK, D), jnp.float32),
            pltpu.VMEM((CBUF,), jnp.float32),
            pltpu.VMEM((CBUF,), jnp.float32),
            pltpu.VMEM((16, 256), jnp.int32),
            pltpu.VMEM((256,), jnp.int32),
        ],
    )
    return f(inputs)


# speedup vs baseline: 1.1079x; 1.1079x over previous
import functools

import numpy as np

import jax
import jax.numpy as jnp
from jax import lax
from jax.experimental import pallas as pl
from jax.experimental.pallas import tpu as pltpu
from jax.experimental.pallas import tpu_sc as plsc

N = 32768
D = 2048
K = 1638
L = 16
NC, NS = 2, 16
NW = NC * NS
ROWS_PER_W = N // NW
CHUNK = 8
CBUF = D + 4 * L
INT_MIN = -2147483648


def _splat(val, dtype):
    return jnp.full((L,), val, dtype)


def _unmap(keys):
    bits = jnp.where(keys >= 0, keys, keys ^ 0x7FFFFFFF)
    return lax.bitcast_convert_type(bits, jnp.float32)


def _body(x_hbm, o_hbm, xbuf, obuf, cbuf1, cbuf2, hist, combo):
    cid = lax.axis_index("c")
    sid = lax.axis_index("s")
    wid = sid * NC + cid
    base_row = wid * ROWS_PER_W
    kk = _splat(K, jnp.int32)
    one = _splat(1, jnp.int32)
    zi = jnp.zeros((L,), jnp.int32)
    zf = jnp.zeros((L,), jnp.float32)
    nan_v = _splat(jnp.nan, jnp.float32)
    ii = lax.iota(jnp.int32, L)

    def zl(l, c3):
        def zc(c, c4):
            hist[l, pl.ds(c * L, L)] = zi
            return c4

        return lax.fori_loop(0, 16, zc, c3)

    lax.fori_loop(0, 16, zl, 0)

    def count_row(r, t):
        @plsc.parallel_loop(0, D, 4 * L, unroll=2, carry=(zi, zi, zi, zi))
        def accs(off, a):
            vs = [xbuf[r, pl.ds(off + j * L, L)] for j in range(4)]
            return tuple(ai + jnp.where(v >= t, one, zi)
                         for ai, v in zip(a, vs))

        return _splat(jnp.sum(sum(accs)), jnp.int32)

    def count_buf(ref, n_pad, t):
        @plsc.parallel_loop(0, n_pad, 2 * L, unroll=2, carry=(zi, zi))
        def accs(off, a):
            vs = [ref[pl.ds(off + j * L, L)] for j in range(2)]
            return tuple(ai + jnp.where(v >= t, one, zi)
                         for ai, v in zip(a, vs))

        return _splat(jnp.sum(sum(accs)), jnp.int32)

    def probe_bits(count_fn, pfx, ff, rr, b_hi, b_lo):
        def rnd(j, state):
            pfx, ff = state
            cand = pfx + (one << (b_hi - j))
            cnt = count_fn(_unmap(cand))
            ok = cnt >= rr
            return jnp.where(ok, cand, pfx), jnp.where(ok, ff, cnt)

        return lax.fori_loop(0, b_hi - b_lo + 1, rnd, (pfx, ff))

    def _compact_group(src, dst, base, off, t_lo, t_hi, width):
        vs = [src(base + j * L) for j in range(width)]
        ms = [(v >= t_lo) & jnp.logical_not(v >= t_hi) for v in vs]
        pcs = [plsc.all_reduce_population_count(m) for m in ms]
        starts = [pcs[0]]
        for j in range(1, width - 1):
            starts.append(starts[-1] + pcs[j])
        plsc.store_compressed(dst.at[pl.ds(off, L)], vs[0], mask=ms[0])
        for j in range(1, width):
            plsc.store_compressed(
                dst.at[pl.ds(off + starts[j - 1][0], L)], vs[j], mask=ms[j])
        return off + (starts[-1] + pcs[-1])[0]

    def compact_from_row(r, t_lo, t_hi):
        def grp(g, off):
            return _compact_group(lambda i: xbuf[r, pl.ds(i, L)], cbuf1,
                                  g * 4 * L, off, t_lo, t_hi, 4)

        n = lax.fori_loop(0, D // (4 * L), grp, np.int32(0))
        for j in range(4):
            cbuf1[pl.ds(n + j * L, L)] = nan_v
        return n

    def compact_from_buf(src, dst, n_pad, t_lo, t_hi):
        def grp(g, off):
            return _compact_group(lambda i: src[pl.ds(i, L)], dst,
                                  g * 2 * L, off, t_lo, t_hi, 2)

        n = lax.fori_loop(0, n_pad // (2 * L), grp, np.int32(0))
        dst[pl.ds(n, L)] = nan_v
        dst[pl.ds(n + L, L)] = nan_v
        return n

    def hist16_level(src, n_pad, pfx, rr, shift):
        pfx_hi = pfx + (one << (shift + 4))
        wrapped = pfx_hi == _splat(INT_MIN, jnp.int32)

        @plsc.parallel_loop(0, n_pad, 2 * L, unroll=2)
        def scat(off):
            for j in range(2):
                v = src[pl.ds(off + j * L, L)]
                iv = lax.bitcast_convert_type(v, jnp.int32)
                key = jnp.where(iv >= 0, iv, iv ^ 0x7FFFFFFF)
                m = (key >= pfx) & ((key < pfx_hi) | wrapped)
                bucket = (key >> shift) & 15
                plsc.addupdate_scatter(hist, [ii, bucket], one, mask=m)

        acc = zi
        for l in range(16):
            acc = acc + hist[l, pl.ds(0, L)]
            hist[l, pl.ds(0, L)] = zi
        rev2 = lax.rev(acc, (0,))
        cs2 = plsc.cumsum(rev2)
        jx = plsc.all_reduce_ffs(cs2 >= rr)
        above = jnp.take_along_axis(cs2 - rev2, jx, axis=0)
        bucket = 15 - jx
        return pfx + (bucket << shift), rr - above

    def do_chunk(ci, carry):
        row0 = base_row + ci * CHUNK
        pltpu.sync_copy(x_hbm.at[pl.ds(row0, CHUNK), :], xbuf)

        def do_row(r, c2):
            @plsc.parallel_loop(0, D, 4 * L, unroll=2,
                               carry=(zf, zf, zf, zf))
            def sq_accs(off, sq):
                out = []
                for a, j in zip(sq, range(4)):
                    v = xbuf[r, pl.ds(off + j * L, L)]
                    iv = lax.bitcast_convert_type(v, jnp.int32)
                    key = jnp.where(iv >= 0, iv, iv ^ 0x7FFFFFFF)
                    bucket = (key >> 24) + 128
                    plsc.addupdate_scatter(hist, [ii, bucket], one)
                    out.append(a + v * v)
                return tuple(out)

            sv = _splat(jnp.sum(sum(sq_accs)), jnp.float32)
            ib = lax.bitcast_convert_type(sv, jnp.int32)
            y = lax.bitcast_convert_type(0x5F3759DF - (ib >> 1), jnp.float32)
            for _ in range(4):
                y = y * (1.5 - 0.5 * sv * y * y)
            boost = jnp.exp(K * y)

            def comb(c, sums):
                acc = zi
                for l in range(16):
                    acc = acc + hist[l, pl.ds(c * L, L)]
                    hist[l, pl.ds(c * L, L)] = zi
                combo[pl.ds(c * L, L)] = acc
                return sums + jnp.where(ii == c, _splat(jnp.sum(acc),
                                                        jnp.int32), zi)

            sums = lax.fori_loop(0, 16, comb, zi)
            rev = lax.rev(sums, (0,))
            cs = plsc.cumsum(rev)
            j0 = plsc.all_reduce_ffs(cs >= kk)
            b_above = jnp.take_along_axis(cs - rev, j0, axis=0)
            chunk = combo[pl.ds((15 - j0[0]) * L, L)]
            rev2 = lax.rev(chunk, (0,))
            cs2 = plsc.cumsum(rev2)
            j1 = plsc.all_reduce_ffs(b_above + cs2 >= kk)
            bkt = (15 - j0) * L + (15 - j1)
            b_above = b_above + jnp.take_along_axis(cs2 - rev2, j1, axis=0)
            pfx = (bkt - 128) << 24
            rr = kk - b_above

            n1 = compact_from_row(r, _unmap(pfx), _unmap(pfx + (1 << 24)))
            n1_pad = ((n1 + 2 * L - 1) // (2 * L)) * (2 * L)

            pfx, rr = hist16_level(cbuf1, n1_pad, pfx, rr, 20)
            n2 = compact_from_buf(cbuf1, cbuf2, n1_pad, _unmap(pfx),
                                  _unmap(pfx + (1 << 20)))
            n2_pad = ((n2 + 2 * L - 1) // (2 * L)) * (2 * L)
            pfx, rr = hist16_level(cbuf2, n2_pad, pfx, rr, 16)
            n3 = compact_from_buf(cbuf2, cbuf1, n2_pad, _unmap(pfx),
                                  _unmap(pfx + (1 << 16)))
            n3_pad = ((n3 + 2 * L - 1) // (2 * L)) * (2 * L)

            def l2_sort(_):
                v = cbuf1[pl.ds(0, L)]
                m = lax.iota(jnp.int32, L) < n3
                sk, _sv, _m = plsc.sort_key_val(v, v, mask=m,
                                                descending=True)
                return jnp.take_along_axis(sk, rr - 1, axis=0)

            def l2_probe(_):
                pfx2, _f = probe_bits(
                    lambda t: count_buf(cbuf1, n3_pad, t), pfx, zi, rr,
                    15, 0)
                return _unmap(pfx2)

            t = lax.cond(n3 <= L, l2_sort, l2_probe, 0)

            @plsc.parallel_loop(0, D, 4 * L, unroll=2)
            def mask_store(off):
                for j in range(4):
                    v = xbuf[r, pl.ds(off + j * L, L)]
                    obuf[r, pl.ds(off + j * L, L)] = jnp.where(
                        v >= t, v * boost, 0.0)

            return c2

        carry = lax.fori_loop(0, CHUNK, do_row, carry)
        pltpu.sync_copy(obuf, o_hbm.at[pl.ds(row0, CHUNK), :])
        return carry

    lax.fori_loop(0, ROWS_PER_W // CHUNK, do_chunk, 0)


@jax.jit
def kernel(inputs):
    f = pl.kernel(
        _body,
        out_type=jax.ShapeDtypeStruct((N, D), jnp.float32),
        mesh=plsc.VectorSubcoreMesh(core_axis_name="c", subcore_axis_name="s"),
        compiler_params=pltpu.CompilerParams(needs_layout_passes=False),
        scratch_types=[
            pltpu.VMEM((CHUNK, D), jnp.float32),
            pltpu.VMEM((CHUNK, D), jnp.float32),
            pltpu.VMEM((CBUF,), jnp.float32),
            pltpu.VMEM((CBUF,), jnp.float32),
            pltpu.VMEM((16, 256), jnp.int32),
            pltpu.VMEM((256,), jnp.int32),
        ],
    )
    return f(inputs)


# speedup vs baseline: 1.2228x; 1.1037x over previous
import functools

import numpy as np

import jax
import jax.numpy as jnp
from jax import lax
from jax.experimental import pallas as pl
from jax.experimental.pallas import tpu as pltpu
from jax.experimental.pallas import tpu_sc as plsc

N = 32768
D = 2048
K = 1638
L = 16
NC, NS = 2, 16
NW = NC * NS
ROWS_PER_W = N // NW
CHUNK = 8
CBUF = D + 4 * L
INT_MIN = -2147483648


def _splat(val, dtype):
    return jnp.full((L,), val, dtype)


def _unmap(keys):
    bits = jnp.where(keys >= 0, keys, keys ^ 0x7FFFFFFF)
    return lax.bitcast_convert_type(bits, jnp.float32)


def _body(x_hbm, o_hbm, xbufA, xbufB, obufA, obufB, cbuf1, cbuf2, hist,
          combo, sinA, sinB, soutA, soutB):
    cid = lax.axis_index("c")
    sid = lax.axis_index("s")
    wid = sid * NC + cid
    base_row = wid * ROWS_PER_W
    kk = _splat(K, jnp.int32)
    one = _splat(1, jnp.int32)
    zi = jnp.zeros((L,), jnp.int32)
    zf = jnp.zeros((L,), jnp.float32)
    nan_v = _splat(jnp.nan, jnp.float32)
    ii = lax.iota(jnp.int32, L)

    def zl(l, c3):
        def zc(c, c4):
            hist[l, pl.ds(c * L, L)] = zi
            return c4

        return lax.fori_loop(0, 16, zc, c3)

    lax.fori_loop(0, 16, zl, 0)

    def count_row(r, t):
        @plsc.parallel_loop(0, D, 4 * L, unroll=2, carry=(zi, zi, zi, zi))
        def accs(off, a):
            vs = [xbuf[r, pl.ds(off + j * L, L)] for j in range(4)]
            return tuple(ai + jnp.where(v >= t, one, zi)
                         for ai, v in zip(a, vs))

        return _splat(jnp.sum(sum(accs)), jnp.int32)

    def count_buf(ref, n_pad, t):
        @plsc.parallel_loop(0, n_pad, 2 * L, unroll=2, carry=(zi, zi))
        def accs(off, a):
            vs = [ref[pl.ds(off + j * L, L)] for j in range(2)]
            return tuple(ai + jnp.where(v >= t, one, zi)
                         for ai, v in zip(a, vs))

        return _splat(jnp.sum(sum(accs)), jnp.int32)

    def probe_bits(count_fn, pfx, ff, rr, b_hi, b_lo):
        def rnd(j, state):
            pfx, ff = state
            cand = pfx + (one << (b_hi - j))
            cnt = count_fn(_unmap(cand))
            ok = cnt >= rr
            return jnp.where(ok, cand, pfx), jnp.where(ok, ff, cnt)

        return lax.fori_loop(0, b_hi - b_lo + 1, rnd, (pfx, ff))

    def _compact_group(src, dst, base, off, t_lo, t_hi, width):
        vs = [src(base + j * L) for j in range(width)]
        ms = [(v >= t_lo) & jnp.logical_not(v >= t_hi) for v in vs]
        pcs = [plsc.all_reduce_population_count(m) for m in ms]
        starts = [pcs[0]]
        for j in range(1, width - 1):
            starts.append(starts[-1] + pcs[j])
        plsc.store_compressed(dst.at[pl.ds(off, L)], vs[0], mask=ms[0])
        for j in range(1, width):
            plsc.store_compressed(
                dst.at[pl.ds(off + starts[j - 1][0], L)], vs[j], mask=ms[j])
        return off + (starts[-1] + pcs[-1])[0]

    def compact_from_row(xbuf, r, t_lo, t_hi):
        def grp(g, off):
            return _compact_group(lambda i: xbuf[r, pl.ds(i, L)], cbuf1,
                                  g * 4 * L, off, t_lo, t_hi, 4)

        n = lax.fori_loop(0, D // (4 * L), grp, np.int32(0))
        for j in range(4):
            cbuf1[pl.ds(n + j * L, L)] = nan_v
        return n

    def compact_from_buf(src, dst, n_pad, t_lo, t_hi):
        def grp(g, off):
            return _compact_group(lambda i: src[pl.ds(i, L)], dst,
                                  g * 2 * L, off, t_lo, t_hi, 2)

        n = lax.fori_loop(0, n_pad // (2 * L), grp, np.int32(0))
        dst[pl.ds(n, L)] = nan_v
        dst[pl.ds(n + L, L)] = nan_v
        return n

    def hist16_level(src, n_pad, pfx, rr, shift):
        pfx_hi = pfx + (one << (shift + 4))
        wrapped = pfx_hi == _splat(INT_MIN, jnp.int32)

        @plsc.parallel_loop(0, n_pad, 2 * L, unroll=2)
        def scat(off):
            for j in range(2):
                v = src[pl.ds(off + j * L, L)]
                iv = lax.bitcast_convert_type(v, jnp.int32)
                key = jnp.where(iv >= 0, iv, iv ^ 0x7FFFFFFF)
                m = (key >= pfx) & ((key < pfx_hi) | wrapped)
                bucket = (key >> shift) & 15
                plsc.addupdate_scatter(hist, [ii, bucket], one, mask=m)

        acc = zi
        for l in range(16):
            acc = acc + hist[l, pl.ds(0, L)]
            hist[l, pl.ds(0, L)] = zi
        rev2 = lax.rev(acc, (0,))
        cs2 = plsc.cumsum(rev2)
        jx = plsc.all_reduce_ffs(cs2 >= rr)
        above = jnp.take_along_axis(cs2 - rev2, jx, axis=0)
        bucket = 15 - jx
        return pfx + (bucket << shift), rr - above

    def in_copy(ci, xb, sem):
        return pltpu.make_async_copy(
            x_hbm.at[pl.ds(base_row + ci * CHUNK, CHUNK), :], xb, sem)

    def out_copy(ci, ob, sem):
        return pltpu.make_async_copy(
            ob, o_hbm.at[pl.ds(base_row + ci * CHUNK, CHUNK), :], sem)

    def do_chunk(ci, xbuf, obuf):
        def do_row(r, c2):
            @plsc.parallel_loop(0, D, 4 * L, unroll=2,
                               carry=(zf, zf, zf, zf))
            def sq_accs(off, sq):
                out = []
                for a, j in zip(sq, range(4)):
                    v = xbuf[r, pl.ds(off + j * L, L)]
                    iv = lax.bitcast_convert_type(v, jnp.int32)
                    key = jnp.where(iv >= 0, iv, iv ^ 0x7FFFFFFF)
                    bucket = (key >> 24) + 128
                    plsc.addupdate_scatter(hist, [ii, bucket], one)
                    out.append(a + v * v)
                return tuple(out)

            sv = _splat(jnp.sum(sum(sq_accs)), jnp.float32)
            ib = lax.bitcast_convert_type(sv, jnp.int32)
            y = lax.bitcast_convert_type(0x5F3759DF - (ib >> 1), jnp.float32)
            for _ in range(4):
                y = y * (1.5 - 0.5 * sv * y * y)
            boost = jnp.exp(K * y)

            def comb(c, sums):
                acc = zi
                for l in range(16):
                    acc = acc + hist[l, pl.ds(c * L, L)]
                    hist[l, pl.ds(c * L, L)] = zi
                combo[pl.ds(c * L, L)] = acc
                return sums + jnp.where(ii == c, _splat(jnp.sum(acc),
                                                        jnp.int32), zi)

            sums = lax.fori_loop(0, 16, comb, zi)
            rev = lax.rev(sums, (0,))
            cs = plsc.cumsum(rev)
            j0 = plsc.all_reduce_ffs(cs >= kk)
            b_above = jnp.take_along_axis(cs - rev, j0, axis=0)
            chunk = combo[pl.ds((15 - j0[0]) * L, L)]
            rev2 = lax.rev(chunk, (0,))
            cs2 = plsc.cumsum(rev2)
            j1 = plsc.all_reduce_ffs(b_above + cs2 >= kk)
            bkt = (15 - j0) * L + (15 - j1)
            b_above = b_above + jnp.take_along_axis(cs2 - rev2, j1, axis=0)
            pfx = (bkt - 128) << 24
            rr = kk - b_above

            n1 = compact_from_row(xbuf, r, _unmap(pfx),
                                  _unmap(pfx + (1 << 24)))
            n1_pad = ((n1 + 2 * L - 1) // (2 * L)) * (2 * L)

            pfx, rr = hist16_level(cbuf1, n1_pad, pfx, rr, 20)
            n2 = compact_from_buf(cbuf1, cbuf2, n1_pad, _unmap(pfx),
                                  _unmap(pfx + (1 << 20)))
            n2_pad = ((n2 + 2 * L - 1) // (2 * L)) * (2 * L)
            pfx, rr = hist16_level(cbuf2, n2_pad, pfx, rr, 16)
            n3 = compact_from_buf(cbuf2, cbuf1, n2_pad, _unmap(pfx),
                                  _unmap(pfx + (1 << 16)))
            n3_pad = ((n3 + 2 * L - 1) // (2 * L)) * (2 * L)

            def l2_sort(_):
                v = cbuf1[pl.ds(0, L)]
                m = lax.iota(jnp.int32, L) < n3
                sk, _sv, _m = plsc.sort_key_val(v, v, mask=m,
                                                descending=True)
                return jnp.take_along_axis(sk, rr - 1, axis=0)

            def l2_probe(_):
                pfx2, _f = probe_bits(
                    lambda t: count_buf(cbuf1, n3_pad, t), pfx, zi, rr,
                    15, 0)
                return _unmap(pfx2)

            t = lax.cond(n3 <= L, l2_sort, l2_probe, 0)

            @plsc.parallel_loop(0, D, 4 * L, unroll=2)
            def mask_store(off):
                for j in range(4):
                    v = xbuf[r, pl.ds(off + j * L, L)]
                    obuf[r, pl.ds(off + j * L, L)] = jnp.where(
                        v >= t, v * boost, 0.0)

            return c2

        lax.fori_loop(0, CHUNK, do_row, 0)

    nch = ROWS_PER_W // CHUNK
    slots = ((xbufA, obufA, sinA, soutA), (xbufB, obufB, sinB, soutB))
    in_copy(0, xbufA, sinA).start()

    def pair(ci2, carry):
        ci = ci2 * 2
        for s in range(2):
            cj = ci + s
            xb, ob, sin, sout = slots[s]
            nxb, _, nsin, _ = slots[1 - s]
            in_copy(cj, xb, sin).wait()

            @pl.when(cj + 1 < nch)
            def _():
                in_copy(cj + 1, nxb, nsin).start()

            @pl.when(cj >= 2)
            def _():
                out_copy(cj - 2, ob, sout).wait()

            do_chunk(cj, xb, ob)
            out_copy(cj, ob, sout).start()
        return carry

    lax.fori_loop(0, nch // 2, pair, 0)
    out_copy(nch - 2, obufA, soutA).wait()
    out_copy(nch - 1, obufB, soutB).wait()


@jax.jit
def kernel(inputs):
    f = pl.kernel(
        _body,
        out_type=jax.ShapeDtypeStruct((N, D), jnp.float32),
        mesh=plsc.VectorSubcoreMesh(core_axis_name="c", subcore_axis_name="s"),
        compiler_params=pltpu.CompilerParams(needs_layout_passes=False),
        scratch_types=[
            pltpu.VMEM((CHUNK, D), jnp.float32),
            pltpu.VMEM((CHUNK, D), jnp.float32),
            pltpu.VMEM((CHUNK, D), jnp.float32),
            pltpu.VMEM((CHUNK, D), jnp.float32),
            pltpu.VMEM((CBUF,), jnp.float32),
            pltpu.VMEM((CBUF,), jnp.float32),
            pltpu.VMEM((16, 256), jnp.int32),
            pltpu.VMEM((256,), jnp.int32),
            pltpu.SemaphoreType.DMA,
            pltpu.SemaphoreType.DMA,
            pltpu.SemaphoreType.DMA,
            pltpu.SemaphoreType.DMA,
        ],
    )
    return f(inputs)


# speedup vs baseline: 1.4838x; 1.2135x over previous
import functools

import numpy as np

import jax
import jax.numpy as jnp
from jax import lax
from jax.experimental import pallas as pl
from jax.experimental.pallas import tpu as pltpu
from jax.experimental.pallas import tpu_sc as plsc

N = 32768
D = 2048
K = 1638
L = 16
NC, NS = 2, 16
NW = NC * NS
ROWS_PER_W = N // NW
CHUNK = 8
CBUF = D + 4 * L
INT_MIN = -2147483648


def _splat(val, dtype):
    return jnp.full((L,), val, dtype)


def _unmap(keys):
    bits = jnp.where(keys >= 0, keys, keys ^ 0x7FFFFFFF)
    return lax.bitcast_convert_type(bits, jnp.float32)


def _body(x_hbm, o_hbm, xbufA, xbufB, obufA, obufB, cbuf1, cbuf2, histv,
          combo, sinA, sinB, soutA, soutB):
    cid = lax.axis_index("c")
    sid = lax.axis_index("s")
    wid = sid * NC + cid
    base_row = wid * ROWS_PER_W
    kk = _splat(K, jnp.int32)
    one = _splat(1, jnp.int32)
    zi = jnp.zeros((L,), jnp.int32)
    zf = jnp.zeros((L,), jnp.float32)
    nan_v = _splat(jnp.nan, jnp.float32)
    ii = lax.iota(jnp.int32, L)

    def zc(c, c4):
        histv[pl.ds(c * L, L)] = zi
        return c4

    lax.fori_loop(0, 16, zc, 0)

    def count_row(r, t):
        @plsc.parallel_loop(0, D, 4 * L, unroll=2, carry=(zi, zi, zi, zi))
        def accs(off, a):
            vs = [xbuf[r, pl.ds(off + j * L, L)] for j in range(4)]
            return tuple(ai + jnp.where(v >= t, one, zi)
                         for ai, v in zip(a, vs))

        return _splat(jnp.sum(sum(accs)), jnp.int32)

    def count_buf(ref, n_pad, t):
        @plsc.parallel_loop(0, n_pad, 2 * L, unroll=2, carry=(zi, zi))
        def accs(off, a):
            vs = [ref[pl.ds(off + j * L, L)] for j in range(2)]
            return tuple(ai + jnp.where(v >= t, one, zi)
                         for ai, v in zip(a, vs))

        return _splat(jnp.sum(sum(accs)), jnp.int32)

    def probe_bits(count_fn, pfx, ff, rr, b_hi, b_lo):
        def rnd(j, state):
            pfx, ff = state
            cand = pfx + (one << (b_hi - j))
            cnt = count_fn(_unmap(cand))
            ok = cnt >= rr
            return jnp.where(ok, cand, pfx), jnp.where(ok, ff, cnt)

        return lax.fori_loop(0, b_hi - b_lo + 1, rnd, (pfx, ff))

    def _compact_group(src, dst, base, off, t_lo, t_hi, width):
        vs = [src(base + j * L) for j in range(width)]
        ms = [(v >= t_lo) & jnp.logical_not(v >= t_hi) for v in vs]
        pcs = [plsc.all_reduce_population_count(m) for m in ms]
        starts = [pcs[0]]
        for j in range(1, width - 1):
            starts.append(starts[-1] + pcs[j])
        plsc.store_compressed(dst.at[pl.ds(off, L)], vs[0], mask=ms[0])
        for j in range(1, width):
            plsc.store_compressed(
                dst.at[pl.ds(off + starts[j - 1][0], L)], vs[j], mask=ms[j])
        return off + (starts[-1] + pcs[-1])[0]

    def compact_from_row(xbuf, r, t_lo, t_hi):
        def grp(g, off):
            return _compact_group(lambda i: xbuf[r, pl.ds(i, L)], cbuf1,
                                  g * 4 * L, off, t_lo, t_hi, 4)

        n = lax.fori_loop(0, D // (4 * L), grp, np.int32(0))
        for j in range(4):
            cbuf1[pl.ds(n + j * L, L)] = nan_v
        return n

    def compact_from_buf(src, dst, n_pad, t_lo, t_hi):
        def grp(g, off):
            return _compact_group(lambda i: src[pl.ds(i, L)], dst,
                                  g * 2 * L, off, t_lo, t_hi, 2)

        n = lax.fori_loop(0, n_pad // (2 * L), grp, np.int32(0))
        dst[pl.ds(n, L)] = nan_v
        dst[pl.ds(n + L, L)] = nan_v
        return n

    def hist16_level(src, n_pad, pfx, rr, shift):
        pfx_hi = pfx + (one << (shift + 4))
        wrapped = pfx_hi == _splat(INT_MIN, jnp.int32)

        @plsc.parallel_loop(0, n_pad, 2 * L, unroll=2)
        def scat(off):
            for j in range(2):
                v = src[pl.ds(off + j * L, L)]
                iv = lax.bitcast_convert_type(v, jnp.int32)
                key = jnp.where(iv >= 0, iv, iv ^ 0x7FFFFFFF)
                m = (key >= pfx) & ((key < pfx_hi) | wrapped)
                bucket = (key >> shift) & 15
                cnts, lm = plsc.scan_count(bucket, mask=m)
                plsc.addupdate_scatter(histv, [bucket], cnts, mask=lm)

        acc = histv[pl.ds(0, L)]
        histv[pl.ds(0, L)] = zi
        rev2 = lax.rev(acc, (0,))
        cs2 = plsc.cumsum(rev2)
        jx = plsc.all_reduce_ffs(cs2 >= rr)
        above = jnp.take_along_axis(cs2 - rev2, jx, axis=0)
        bucket = 15 - jx
        return pfx + (bucket << shift), rr - above

    def in_copy(ci, xb, sem):
        return pltpu.make_async_copy(
            x_hbm.at[pl.ds(base_row + ci * CHUNK, CHUNK), :], xb, sem)

    def out_copy(ci, ob, sem):
        return pltpu.make_async_copy(
            ob, o_hbm.at[pl.ds(base_row + ci * CHUNK, CHUNK), :], sem)

    def do_chunk(ci, xbuf, obuf):
        def do_row(r, c2):
            @plsc.parallel_loop(0, D, 4 * L, unroll=2,
                               carry=(zf, zf, zf, zf))
            def sq_accs(off, sq):
                out = []
                for a, j in zip(sq, range(4)):
                    v = xbuf[r, pl.ds(off + j * L, L)]
                    iv = lax.bitcast_convert_type(v, jnp.int32)
                    key = jnp.where(iv >= 0, iv, iv ^ 0x7FFFFFFF)
                    bucket = (key >> 24) + 128
                    cnts, lm = plsc.scan_count(bucket)
                    plsc.addupdate_scatter(histv, [bucket], cnts, mask=lm)
                    out.append(a + v * v)
                return tuple(out)

            sv = _splat(jnp.sum(sum(sq_accs)), jnp.float32)
            ib = lax.bitcast_convert_type(sv, jnp.int32)
            y = lax.bitcast_convert_type(0x5F3759DF - (ib >> 1), jnp.float32)
            for _ in range(4):
                y = y * (1.5 - 0.5 * sv * y * y)
            boost = jnp.exp(K * y)

            def comb(c, sums):
                acc = histv[pl.ds(c * L, L)]
                histv[pl.ds(c * L, L)] = zi
                combo[pl.ds(c * L, L)] = acc
                return sums + jnp.where(ii == c, _splat(jnp.sum(acc),
                                                        jnp.int32), zi)

            sums = lax.fori_loop(0, 16, comb, zi)
            rev = lax.rev(sums, (0,))
            cs = plsc.cumsum(rev)
            j0 = plsc.all_reduce_ffs(cs >= kk)
            b_above = jnp.take_along_axis(cs - rev, j0, axis=0)
            chunk = combo[pl.ds((15 - j0[0]) * L, L)]
            rev2 = lax.rev(chunk, (0,))
            cs2 = plsc.cumsum(rev2)
            j1 = plsc.all_reduce_ffs(b_above + cs2 >= kk)
            bkt = (15 - j0) * L + (15 - j1)
            b_above = b_above + jnp.take_along_axis(cs2 - rev2, j1, axis=0)
            pfx = (bkt - 128) << 24
            rr = kk - b_above

            n1 = compact_from_row(xbuf, r, _unmap(pfx),
                                  _unmap(pfx + (1 << 24)))
            n1_pad = ((n1 + 2 * L - 1) // (2 * L)) * (2 * L)

            pfx, rr = hist16_level(cbuf1, n1_pad, pfx, rr, 20)
            n2 = compact_from_buf(cbuf1, cbuf2, n1_pad, _unmap(pfx),
                                  _unmap(pfx + (1 << 20)))
            n2_pad = ((n2 + 2 * L - 1) // (2 * L)) * (2 * L)
            pfx, rr = hist16_level(cbuf2, n2_pad, pfx, rr, 16)
            n3 = compact_from_buf(cbuf2, cbuf1, n2_pad, _unmap(pfx),
                                  _unmap(pfx + (1 << 16)))
            n3_pad = ((n3 + 2 * L - 1) // (2 * L)) * (2 * L)

            def l2_sort(_):
                v = cbuf1[pl.ds(0, L)]
                m = lax.iota(jnp.int32, L) < n3
                sk, _sv, _m = plsc.sort_key_val(v, v, mask=m,
                                                descending=True)
                return jnp.take_along_axis(sk, rr - 1, axis=0)

            def l2_probe(_):
                pfx2, _f = probe_bits(
                    lambda t: count_buf(cbuf1, n3_pad, t), pfx, zi, rr,
                    15, 0)
                return _unmap(pfx2)

            t = lax.cond(n3 <= L, l2_sort, l2_probe, 0)

            @plsc.parallel_loop(0, D, 4 * L, unroll=2)
            def mask_store(off):
                for j in range(4):
                    v = xbuf[r, pl.ds(off + j * L, L)]
                    obuf[r, pl.ds(off + j * L, L)] = jnp.where(
                        v >= t, v * boost, 0.0)

            return c2

        lax.fori_loop(0, CHUNK, do_row, 0)

    nch = ROWS_PER_W // CHUNK
    slots = ((xbufA, obufA, sinA, soutA), (xbufB, obufB, sinB, soutB))
    in_copy(0, xbufA, sinA).start()

    def pair(ci2, carry):
        ci = ci2 * 2
        for s in range(2):
            cj = ci + s
            xb, ob, sin, sout = slots[s]
            nxb, _, nsin, _ = slots[1 - s]
            in_copy(cj, xb, sin).wait()

            @pl.when(cj + 1 < nch)
            def _():
                in_copy(cj + 1, nxb, nsin).start()

            @pl.when(cj >= 2)
            def _():
                out_copy(cj - 2, ob, sout).wait()

            do_chunk(cj, xb, ob)
            out_copy(cj, ob, sout).start()
        return carry

    lax.fori_loop(0, nch // 2, pair, 0)
    out_copy(nch - 2, obufA, soutA).wait()
    out_copy(nch - 1, obufB, soutB).wait()


@jax.jit
def kernel(inputs):
    f = pl.kernel(
        _body,
        out_type=jax.ShapeDtypeStruct((N, D), jnp.float32),
        mesh=plsc.VectorSubcoreMesh(core_axis_name="c", subcore_axis_name="s"),
        compiler_params=pltpu.CompilerParams(needs_layout_passes=False),
        scratch_types=[
            pltpu.VMEM((CHUNK, D), jnp.float32),
            pltpu.VMEM((CHUNK, D), jnp.float32),
            pltpu.VMEM((CHUNK, D), jnp.float32),
            pltpu.VMEM((CHUNK, D), jnp.float32),
            pltpu.VMEM((CBUF,), jnp.float32),
            pltpu.VMEM((CBUF,), jnp.float32),
            pltpu.VMEM((256,), jnp.int32),
            pltpu.VMEM((256,), jnp.int32),
            pltpu.SemaphoreType.DMA,
            pltpu.SemaphoreType.DMA,
            pltpu.SemaphoreType.DMA,
            pltpu.SemaphoreType.DMA,
        ],
    )
    return f(inputs)


# speedup vs baseline: 2.1391x; 1.4416x over previous
import functools

import numpy as np

import jax
import jax.numpy as jnp
from jax import lax
from jax.experimental import pallas as pl
from jax.experimental.pallas import tpu as pltpu
from jax.experimental.pallas import tpu_sc as plsc

N = 32768
D = 2048
K = 1638
L = 16
NC, NS = 2, 16
NW = NC * NS
ROWS_PER_W = N // NW
CHUNK = 8
CBUF = D + 4 * L
INT_MIN = -2147483648


def _splat(val, dtype):
    return jnp.full((L,), val, dtype)


def _unmap(keys):
    bits = jnp.where(keys >= 0, keys, keys ^ 0x7FFFFFFF)
    return lax.bitcast_convert_type(bits, jnp.float32)


def _body(x_hbm, o_hbm, xbufA, xbufB, obufA, obufB, cbuf1, cbuf2, histv,
          combo, sinA, sinB, soutA, soutB):
    cid = lax.axis_index("c")
    sid = lax.axis_index("s")
    wid = sid * NC + cid
    base_row = wid * ROWS_PER_W
    kk = _splat(K, jnp.int32)
    one = _splat(1, jnp.int32)
    zi = jnp.zeros((L,), jnp.int32)
    zf = jnp.zeros((L,), jnp.float32)
    nan_v = _splat(jnp.nan, jnp.float32)
    ii = lax.iota(jnp.int32, L)

    def zc(c, c4):
        histv[pl.ds(c * L, L)] = zi
        return c4

    lax.fori_loop(0, 16, zc, 0)

    def count_row(r, t):
        @plsc.parallel_loop(0, D, 4 * L, unroll=2, carry=(zi, zi, zi, zi))
        def accs(off, a):
            vs = [xbuf[r, pl.ds(off + j * L, L)] for j in range(4)]
            return tuple(ai + jnp.where(v >= t, one, zi)
                         for ai, v in zip(a, vs))

        return _splat(jnp.sum(sum(accs)), jnp.int32)

    def count_buf(ref, n_pad, t):
        @plsc.parallel_loop(0, n_pad, 2 * L, unroll=2, carry=(zi, zi))
        def accs(off, a):
            vs = [ref[pl.ds(off + j * L, L)] for j in range(2)]
            return tuple(ai + jnp.where(v >= t, one, zi)
                         for ai, v in zip(a, vs))

        return _splat(jnp.sum(sum(accs)), jnp.int32)

    def probe_bits(count_fn, pfx, ff, rr, b_hi, b_lo):
        def rnd(j, state):
            pfx, ff = state
            cand = pfx + (one << (b_hi - j))
            cnt = count_fn(_unmap(cand))
            ok = cnt >= rr
            return jnp.where(ok, cand, pfx), jnp.where(ok, ff, cnt)

        return lax.fori_loop(0, b_hi - b_lo + 1, rnd, (pfx, ff))

    def _compact_group(src, dst, base, off, t_lo, t_hi, width):
        vs = [src(base + j * L) for j in range(width)]
        ms = [(v >= t_lo) & jnp.logical_not(v >= t_hi) for v in vs]
        pcs = [plsc.all_reduce_population_count(m) for m in ms]
        starts = [pcs[0]]
        for j in range(1, width - 1):
            starts.append(starts[-1] + pcs[j])
        plsc.store_compressed(dst.at[pl.ds(off, L)], vs[0], mask=ms[0])
        for j in range(1, width):
            plsc.store_compressed(
                dst.at[pl.ds(off + starts[j - 1][0], L)], vs[j], mask=ms[j])
        return off + (starts[-1] + pcs[-1])[0]

    def compact_from_row(xbuf, r, t_lo, t_hi):
        @plsc.parallel_loop(0, D, 4 * L, unroll=2, carry=jnp.zeros((), jnp.int32))
        def n(i, off):
            return _compact_group(lambda w: xbuf[r, pl.ds(w, L)], cbuf1,
                                  i, off, t_lo, t_hi, 4)

        for j in range(4):
            cbuf1[pl.ds(n + j * L, L)] = nan_v
        return n

    def compact_from_buf(src, dst, n_pad, t_lo, t_hi):
        @plsc.parallel_loop(0, n_pad, 2 * L, unroll=2, carry=jnp.zeros((), jnp.int32))
        def n(i, off):
            return _compact_group(lambda w: src[pl.ds(w, L)], dst,
                                  i, off, t_lo, t_hi, 2)

        dst[pl.ds(n, L)] = nan_v
        dst[pl.ds(n + L, L)] = nan_v
        return n

    def hist16_level(src, n_pad, pfx, rr, shift):
        pfx_hi = pfx + (one << (shift + 4))
        wrapped = pfx_hi == _splat(INT_MIN, jnp.int32)

        @plsc.parallel_loop(0, n_pad, 2 * L, unroll=2)
        def scat(off):
            for j in range(2):
                v = src[pl.ds(off + j * L, L)]
                iv = lax.bitcast_convert_type(v, jnp.int32)
                key = jnp.where(iv >= 0, iv, iv ^ 0x7FFFFFFF)
                m = (key >= pfx) & ((key < pfx_hi) | wrapped)
                bucket = (key >> shift) & 15
                cnts, lm = plsc.scan_count(bucket, mask=m)
                plsc.addupdate_scatter(histv, [bucket], cnts, mask=lm)

        acc = histv[pl.ds(0, L)]
        histv[pl.ds(0, L)] = zi
        rev2 = lax.rev(acc, (0,))
        cs2 = plsc.cumsum(rev2)
        jx = plsc.all_reduce_ffs(cs2 >= rr)
        above = jnp.take_along_axis(cs2 - rev2, jx, axis=0)
        bucket = 15 - jx
        return pfx + (bucket << shift), rr - above

    def in_copy(ci, xb, sem):
        return pltpu.make_async_copy(
            x_hbm.at[pl.ds(base_row + ci * CHUNK, CHUNK), :], xb, sem)

    def out_copy(ci, ob, sem):
        return pltpu.make_async_copy(
            ob, o_hbm.at[pl.ds(base_row + ci * CHUNK, CHUNK), :], sem)

    def do_chunk(ci, xbuf, obuf):
        def do_row(r, c2):
            @plsc.parallel_loop(0, D, 4 * L, unroll=2,
                               carry=(zf, zf, zf, zf))
            def sq_accs(off, sq):
                out = []
                for a, j in zip(sq, range(4)):
                    v = xbuf[r, pl.ds(off + j * L, L)]
                    iv = lax.bitcast_convert_type(v, jnp.int32)
                    key = jnp.where(iv >= 0, iv, iv ^ 0x7FFFFFFF)
                    bucket = (key >> 24) + 128
                    cnts, lm = plsc.scan_count(bucket)
                    plsc.addupdate_scatter(histv, [bucket], cnts, mask=lm)
                    out.append(a + v * v)
                return tuple(out)

            sv = _splat(jnp.sum(sum(sq_accs)), jnp.float32)
            ib = lax.bitcast_convert_type(sv, jnp.int32)
            y = lax.bitcast_convert_type(0x5F3759DF - (ib >> 1), jnp.float32)
            for _ in range(4):
                y = y * (1.5 - 0.5 * sv * y * y)
            boost = jnp.exp(K * y)

            parts = []
            for c in range(16):
                acc = histv[pl.ds(c * L, L)]
                histv[pl.ds(c * L, L)] = zi
                combo[pl.ds(c * L, L)] = acc
                parts.append(jnp.where(ii == c,
                                       _splat(jnp.sum(acc), jnp.int32), zi))
            sums = sum(parts)
            rev = lax.rev(sums, (0,))
            cs = plsc.cumsum(rev)
            j0 = plsc.all_reduce_ffs(cs >= kk)
            b_above = jnp.take_along_axis(cs - rev, j0, axis=0)
            chunk = combo[pl.ds((15 - j0[0]) * L, L)]
            rev2 = lax.rev(chunk, (0,))
            cs2 = plsc.cumsum(rev2)
            j1 = plsc.all_reduce_ffs(b_above + cs2 >= kk)
            bkt = (15 - j0) * L + (15 - j1)
            b_above = b_above + jnp.take_along_axis(cs2 - rev2, j1, axis=0)
            pfx = (bkt - 128) << 24
            rr = kk - b_above

            n1 = compact_from_row(xbuf, r, _unmap(pfx),
                                  _unmap(pfx + (1 << 24)))
            n1_pad = ((n1 + 2 * L - 1) // (2 * L)) * (2 * L)

            pfx, rr = hist16_level(cbuf1, n1_pad, pfx, rr, 20)
            n2 = compact_from_buf(cbuf1, cbuf2, n1_pad, _unmap(pfx),
                                  _unmap(pfx + (1 << 20)))
            n2_pad = ((n2 + 2 * L - 1) // (2 * L)) * (2 * L)
            pfx, rr = hist16_level(cbuf2, n2_pad, pfx, rr, 16)
            n3 = compact_from_buf(cbuf2, cbuf1, n2_pad, _unmap(pfx),
                                  _unmap(pfx + (1 << 16)))
            n3_pad = ((n3 + 2 * L - 1) // (2 * L)) * (2 * L)

            def l2_sort(_):
                v = cbuf1[pl.ds(0, L)]
                m = lax.iota(jnp.int32, L) < n3
                sk, _sv, _m = plsc.sort_key_val(v, v, mask=m,
                                                descending=True)
                return jnp.take_along_axis(sk, rr - 1, axis=0)

            def l2_probe(_):
                pfx2, _f = probe_bits(
                    lambda t: count_buf(cbuf1, n3_pad, t), pfx, zi, rr,
                    15, 0)
                return _unmap(pfx2)

            t = lax.cond(n3 <= L, l2_sort, l2_probe, 0)

            @plsc.parallel_loop(0, D, 4 * L, unroll=2)
            def mask_store(off):
                for j in range(4):
                    v = xbuf[r, pl.ds(off + j * L, L)]
                    obuf[r, pl.ds(off + j * L, L)] = jnp.where(
                        v >= t, v * boost, 0.0)

            return c2

        lax.fori_loop(0, CHUNK, do_row, 0)

    nch = ROWS_PER_W // CHUNK
    slots = ((xbufA, obufA, sinA, soutA), (xbufB, obufB, sinB, soutB))
    in_copy(0, xbufA, sinA).start()

    def pair(ci2, carry):
        ci = ci2 * 2
        for s in range(2):
            cj = ci + s
            xb, ob, sin, sout = slots[s]
            nxb, _, nsin, _ = slots[1 - s]
            in_copy(cj, xb, sin).wait()

            @pl.when(cj + 1 < nch)
            def _():
                in_copy(cj + 1, nxb, nsin).start()

            @pl.when(cj >= 2)
            def _():
                out_copy(cj - 2, ob, sout).wait()

            do_chunk(cj, xb, ob)
            out_copy(cj, ob, sout).start()
        return carry

    lax.fori_loop(0, nch // 2, pair, 0)
    out_copy(nch - 2, obufA, soutA).wait()
    out_copy(nch - 1, obufB, soutB).wait()


@jax.jit
def kernel(inputs):
    f = pl.kernel(
        _body,
        out_type=jax.ShapeDtypeStruct((N, D), jnp.float32),
        mesh=plsc.VectorSubcoreMesh(core_axis_name="c", subcore_axis_name="s"),
        compiler_params=pltpu.CompilerParams(needs_layout_passes=False),
        scratch_types=[
            pltpu.VMEM((CHUNK, D), jnp.float32),
            pltpu.VMEM((CHUNK, D), jnp.float32),
            pltpu.VMEM((CHUNK, D), jnp.float32),
            pltpu.VMEM((CHUNK, D), jnp.float32),
            pltpu.VMEM((CBUF,), jnp.float32),
            pltpu.VMEM((CBUF,), jnp.float32),
            pltpu.VMEM((256,), jnp.int32),
            pltpu.VMEM((256,), jnp.int32),
            pltpu.SemaphoreType.DMA,
            pltpu.SemaphoreType.DMA,
            pltpu.SemaphoreType.DMA,
            pltpu.SemaphoreType.DMA,
        ],
    )
    return f(inputs)


# speedup vs baseline: 2.1930x; 1.0252x over previous
import functools

import numpy as np

import jax
import jax.numpy as jnp
from jax import lax
from jax.experimental import pallas as pl
from jax.experimental.pallas import tpu as pltpu
from jax.experimental.pallas import tpu_sc as plsc

N = 32768
D = 2048
K = 1638
L = 16
NC, NS = 2, 16
NW = NC * NS
ROWS_PER_W = N // NW
CHUNK = 8
CBUF = D + 4 * L
INT_MIN = -2147483648


def _splat(val, dtype):
    return jnp.full((L,), val, dtype)


def _unmap(keys):
    bits = jnp.where(keys >= 0, keys, keys ^ 0x7FFFFFFF)
    return lax.bitcast_convert_type(bits, jnp.float32)


def _body(x_hbm, o_hbm, xbufA, xbufB, obufA, obufB, cbuf1, cbuf2, histv,
          combo, sinA, sinB, soutA, soutB):
    cid = lax.axis_index("c")
    sid = lax.axis_index("s")
    wid = sid * NC + cid
    base_row = wid * ROWS_PER_W
    kk = _splat(K, jnp.int32)
    one = _splat(1, jnp.int32)
    zi = jnp.zeros((L,), jnp.int32)
    zf = jnp.zeros((L,), jnp.float32)
    nan_v = _splat(jnp.nan, jnp.float32)
    ii = lax.iota(jnp.int32, L)

    def zc(c, c4):
        histv[pl.ds(c * L, L)] = zi
        return c4

    lax.fori_loop(0, 16, zc, 0)

    def count_row(r, t):
        @plsc.parallel_loop(0, D, 4 * L, unroll=2, carry=(zi, zi, zi, zi))
        def accs(off, a):
            vs = [xbuf[r, pl.ds(off + j * L, L)] for j in range(4)]
            return tuple(ai + jnp.where(v >= t, one, zi)
                         for ai, v in zip(a, vs))

        return _splat(jnp.sum(sum(accs)), jnp.int32)

    def count_buf(ref, n_pad, t):
        @plsc.parallel_loop(0, n_pad, 2 * L, unroll=2, carry=(zi, zi))
        def accs(off, a):
            vs = [ref[pl.ds(off + j * L, L)] for j in range(2)]
            return tuple(ai + jnp.where(v >= t, one, zi)
                         for ai, v in zip(a, vs))

        return _splat(jnp.sum(sum(accs)), jnp.int32)

    def probe_bits(count_fn, pfx, ff, rr, b_hi, b_lo):
        def rnd(j, state):
            pfx, ff = state
            cand = pfx + (one << (b_hi - j))
            cnt = count_fn(_unmap(cand))
            ok = cnt >= rr
            return jnp.where(ok, cand, pfx), jnp.where(ok, ff, cnt)

        return lax.fori_loop(0, b_hi - b_lo + 1, rnd, (pfx, ff))

    def _compact_group(src, dst, base, off, t_lo, t_hi, width):
        vs = [src(base + j * L) for j in range(width)]
        ms = [(v >= t_lo) & jnp.logical_not(v >= t_hi) for v in vs]
        pcs = [plsc.all_reduce_population_count(m) for m in ms]
        starts = [pcs[0]]
        for j in range(1, width - 1):
            starts.append(starts[-1] + pcs[j])
        plsc.store_compressed(dst.at[pl.ds(off, L)], vs[0], mask=ms[0])
        for j in range(1, width):
            plsc.store_compressed(
                dst.at[pl.ds(off + starts[j - 1][0], L)], vs[j], mask=ms[j])
        return off + (starts[-1] + pcs[-1])[0]

    def compact_from_row(xbuf, r, t_lo, t_hi):
        @plsc.parallel_loop(0, D, 4 * L, unroll=2, carry=jnp.zeros((), jnp.int32))
        def n(i, off):
            return _compact_group(lambda w: xbuf[r, pl.ds(w, L)], cbuf1,
                                  i, off, t_lo, t_hi, 4)

        for j in range(4):
            cbuf1[pl.ds(n + j * L, L)] = nan_v
        return n

    def compact_from_buf(src, dst, n_pad, t_lo, t_hi):
        @plsc.parallel_loop(0, n_pad, 2 * L, unroll=2, carry=jnp.zeros((), jnp.int32))
        def n(i, off):
            return _compact_group(lambda w: src[pl.ds(w, L)], dst,
                                  i, off, t_lo, t_hi, 2)

        dst[pl.ds(n, L)] = nan_v
        dst[pl.ds(n + L, L)] = nan_v
        return n

    def hist16_level(src, n_pad, pfx, rr, shift):
        pfx_hi = pfx + (one << (shift + 4))
        wrapped = pfx_hi == _splat(INT_MIN, jnp.int32)

        @plsc.parallel_loop(0, n_pad, 2 * L, unroll=2)
        def scat(off):
            for j in range(2):
                v = src[pl.ds(off + j * L, L)]
                iv = lax.bitcast_convert_type(v, jnp.int32)
                key = jnp.where(iv >= 0, iv, iv ^ 0x7FFFFFFF)
                m = (key >= pfx) & ((key < pfx_hi) | wrapped)
                bucket = (key >> shift) & 15
                cnts, lm = plsc.scan_count(bucket, mask=m)
                plsc.addupdate_scatter(histv, [bucket], cnts, mask=lm)

        acc = histv[pl.ds(0, L)]
        histv[pl.ds(0, L)] = zi
        rev2 = lax.rev(acc, (0,))
        cs2 = plsc.cumsum(rev2)
        jx = plsc.all_reduce_ffs(cs2 >= rr)
        above = jnp.take_along_axis(cs2 - rev2, jx, axis=0)
        bucket = 15 - jx
        return pfx + (bucket << shift), rr - above

    def in_copy(ci, xb, sem):
        return pltpu.make_async_copy(
            x_hbm.at[pl.ds(base_row + ci * CHUNK, CHUNK), :], xb, sem)

    def out_copy(ci, ob, sem):
        return pltpu.make_async_copy(
            ob, o_hbm.at[pl.ds(base_row + ci * CHUNK, CHUNK), :], sem)

    def do_chunk(ci, xbuf, obuf):
        def do_row(r, c2):
            @plsc.parallel_loop(0, D, 4 * L, unroll=4,
                               carry=(zf, zf, zf, zf))
            def sq_accs(off, sq):
                out = []
                for a, j in zip(sq, range(4)):
                    v = xbuf[r, pl.ds(off + j * L, L)]
                    iv = lax.bitcast_convert_type(v, jnp.int32)
                    key = jnp.where(iv >= 0, iv, iv ^ 0x7FFFFFFF)
                    bucket = (key >> 24) + 128
                    cnts, lm = plsc.scan_count(bucket)
                    plsc.addupdate_scatter(histv, [bucket], cnts, mask=lm)
                    out.append(a + v * v)
                return tuple(out)

            sv = _splat(jnp.sum(sum(sq_accs)), jnp.float32)
            ib = lax.bitcast_convert_type(sv, jnp.int32)
            y = lax.bitcast_convert_type(0x5F3759DF - (ib >> 1), jnp.float32)
            for _ in range(4):
                y = y * (1.5 - 0.5 * sv * y * y)
            boost = jnp.exp(K * y)

            parts = []
            for c in range(16):
                acc = histv[pl.ds(c * L, L)]
                histv[pl.ds(c * L, L)] = zi
                combo[pl.ds(c * L, L)] = acc
                parts.append(jnp.where(ii == c,
                                       _splat(jnp.sum(acc), jnp.int32), zi))
            sums = sum(parts)
            rev = lax.rev(sums, (0,))
            cs = plsc.cumsum(rev)
            j0 = plsc.all_reduce_ffs(cs >= kk)
            b_above = jnp.take_along_axis(cs - rev, j0, axis=0)
            chunk = combo[pl.ds((15 - j0[0]) * L, L)]
            rev2 = lax.rev(chunk, (0,))
            cs2 = plsc.cumsum(rev2)
            j1 = plsc.all_reduce_ffs(b_above + cs2 >= kk)
            bkt = (15 - j0) * L + (15 - j1)
            b_above = b_above + jnp.take_along_axis(cs2 - rev2, j1, axis=0)
            pfx = (bkt - 128) << 24
            rr = kk - b_above

            n1 = compact_from_row(xbuf, r, _unmap(pfx),
                                  _unmap(pfx + (1 << 24)))
            n1_pad = ((n1 + 2 * L - 1) // (2 * L)) * (2 * L)

            pfx, rr = hist16_level(cbuf1, n1_pad, pfx, rr, 20)
            n2 = compact_from_buf(cbuf1, cbuf2, n1_pad, _unmap(pfx),
                                  _unmap(pfx + (1 << 20)))
            n2_pad = ((n2 + 2 * L - 1) // (2 * L)) * (2 * L)
            pfx, rr = hist16_level(cbuf2, n2_pad, pfx, rr, 16)
            n3 = compact_from_buf(cbuf2, cbuf1, n2_pad, _unmap(pfx),
                                  _unmap(pfx + (1 << 16)))
            n3_pad = ((n3 + 2 * L - 1) // (2 * L)) * (2 * L)

            def l2_sort(_):
                v = cbuf1[pl.ds(0, L)]
                m = lax.iota(jnp.int32, L) < n3
                sk, _sv, _m = plsc.sort_key_val(v, v, mask=m,
                                                descending=True)
                return jnp.take_along_axis(sk, rr - 1, axis=0)

            def l2_probe(_):
                pfx2, _f = probe_bits(
                    lambda t: count_buf(cbuf1, n3_pad, t), pfx, zi, rr,
                    15, 0)
                return _unmap(pfx2)

            t = lax.cond(n3 <= L, l2_sort, l2_probe, 0)

            @plsc.parallel_loop(0, D, 4 * L, unroll=4)
            def mask_store(off):
                for j in range(4):
                    v = xbuf[r, pl.ds(off + j * L, L)]
                    obuf[r, pl.ds(off + j * L, L)] = jnp.where(
                        v >= t, v * boost, 0.0)

            return c2

        lax.fori_loop(0, CHUNK, do_row, 0)

    nch = ROWS_PER_W // CHUNK
    slots = ((xbufA, obufA, sinA, soutA), (xbufB, obufB, sinB, soutB))
    in_copy(0, xbufA, sinA).start()

    def pair(ci2, carry):
        ci = ci2 * 2
        for s in range(2):
            cj = ci + s
            xb, ob, sin, sout = slots[s]
            nxb, _, nsin, _ = slots[1 - s]
            in_copy(cj, xb, sin).wait()

            @pl.when(cj + 1 < nch)
            def _():
                in_copy(cj + 1, nxb, nsin).start()

            @pl.when(cj >= 2)
            def _():
                out_copy(cj - 2, ob, sout).wait()

            do_chunk(cj, xb, ob)
            out_copy(cj, ob, sout).start()
        return carry

    lax.fori_loop(0, nch // 2, pair, 0)
    out_copy(nch - 2, obufA, soutA).wait()
    out_copy(nch - 1, obufB, soutB).wait()


@jax.jit
def kernel(inputs):
    f = pl.kernel(
        _body,
        out_type=jax.ShapeDtypeStruct((N, D), jnp.float32),
        mesh=plsc.VectorSubcoreMesh(core_axis_name="c", subcore_axis_name="s"),
        compiler_params=pltpu.CompilerParams(needs_layout_passes=False),
        scratch_types=[
            pltpu.VMEM((CHUNK, D), jnp.float32),
            pltpu.VMEM((CHUNK, D), jnp.float32),
            pltpu.VMEM((CHUNK, D), jnp.float32),
            pltpu.VMEM((CHUNK, D), jnp.float32),
            pltpu.VMEM((CBUF,), jnp.float32),
            pltpu.VMEM((CBUF,), jnp.float32),
            pltpu.VMEM((256,), jnp.int32),
            pltpu.VMEM((256,), jnp.int32),
            pltpu.SemaphoreType.DMA,
            pltpu.SemaphoreType.DMA,
            pltpu.SemaphoreType.DMA,
            pltpu.SemaphoreType.DMA,
        ],
    )
    return f(inputs)
